# baseline (speedup 1.0000x reference)
import functools

import jax
import jax.numpy as jnp
from jax import lax
from jax.experimental import pallas as pl
from jax.experimental.pallas import tpu as pltpu
from jax.experimental.pallas import tpu_sc as plsc

K = 20
KPAD = 24
DP = 128
BR = 256
BRG = 64
GW = 16
NEG = -3.0e38
_ONE_LO = 0x3F7FFFFF

_SPLAT_DNUMS = lax.GatherDimensionNumbers(
    offset_dims=(), collapsed_slice_dims=(0,), start_index_map=(0,))


def _attr_kernel(x_ref, w_ref, b_ref, out_ref):
    z = jnp.dot(x_ref[...], w_ref[...], preferred_element_type=jnp.float32)
    out_ref[...] = jax.nn.sigmoid(z + b_ref[...])


def _maxval_kernel(ab_ref, aall_ref, mv_ref):
    a = jax.lax.dot_general(
        ab_ref[...], aall_ref[...],
        (((1,), (1,)), ((), ())),
        preferred_element_type=jnp.float32,
    )
    pmax = jnp.max(a, axis=0, keepdims=True)

    @pl.when(pl.program_id(0) == 0)
    def _():
        mv_ref[...] = pmax

    @pl.when(pl.program_id(0) != 0)
    def _():
        mv_ref[...] = jnp.maximum(mv_ref[...], pmax)


def _groups_kernel(ab_ref, aall_ref, mv_ref, tab_ref, gidx_ref, *, n, br):
    i = pl.program_id(0)
    ng = n // GW
    a = jax.lax.dot_general(
        ab_ref[...], aall_ref[...],
        (((1,), (1,)), ((), ())),
        preferred_element_type=jnp.float32,
    )
    slab = a / mv_ref[...]
    tab_ref[...] = slab
    gmax = jnp.max(slab.reshape(br, ng, GW), axis=2)
    giota = lax.broadcasted_iota(jnp.int32, (br, ng), 1)
    kiota = lax.broadcasted_iota(jnp.int32, (br, KPAD), 1)
    gacc = jnp.zeros((br, KPAD), jnp.int32)
    for k in range(K):
        m = jnp.max(gmax, axis=1, keepdims=True)
        cand = jnp.where(gmax == m, giota, ng)
        g = jnp.min(cand, axis=1, keepdims=True)
        gmax = jnp.where(cand == g, NEG, gmax)
        gacc = jnp.where(kiota == k, g, gacc)
    gidx_ref[...] = gacc


def _sc_select_kernel(tab_hbm, gidx_hbm, vals_hbm, idx_hbm,
                      gidx_v, row_v, ovals_v, oidx_v, sem,
                      *, rows_per_w, nc, ng):
    wid = lax.axis_index("s") * nc + lax.axis_index("c")
    rb = wid * rows_per_w
    pltpu.sync_copy(gidx_hbm.at[pl.ds(rb, rows_per_w)], gidx_v)
    iota16 = lax.iota(jnp.int32, GW)

    @pl.loop(0, rows_per_w)
    def _(r):
        pltpu.async_copy(tab_hbm.at[rb + r], row_v, sem).wait()
        g0 = gidx_v[r, pl.ds(0, GW)]
        g1 = gidx_v[r, pl.ds(8, GW)]

        hi_k = jnp.full((GW,), NEG, jnp.float32)
        hi_v = jnp.zeros((GW,), jnp.int32)
        lo_k = jnp.full((GW,), NEG, jnp.float32)
        lo_v = jnp.zeros((GW,), jnp.int32)
        for k in range(K):
            gvec = g0 if k < GW else g1
            pos = k if k < GW else k - 8
            gs = lax.gather(
                gvec, jnp.full((GW, 1), pos, jnp.int32), _SPLAT_DNUMS, (1,),
                mode=lax.GatherScatterMode.PROMISE_IN_BOUNDS)
            col = gs * GW + iota16
            v = plsc.load_gather(row_v, [col])
            vb = plsc.bitcast(v, jnp.int32)
            band = vb - _ONE_LO
            tieb = (8192.0 - col.astype(jnp.float32)) * (2.0 ** -13)
            key = jnp.where(band >= 0,
                            2.0 + 2.0 * band.astype(jnp.float32) + tieb, v)
            ks, vs = plsc.sort_key_val(key, col, descending=True)
            rk = lax.rev(ks, (0,))
            rv = lax.rev(vs, (0,))
            m1 = hi_k >= rk
            nk = jnp.where(m1, hi_k, rk)
            nv = jnp.where(m1, hi_v, rv)
            sk = jnp.where(m1, rk, hi_k)
            sv = jnp.where(m1, rv, hi_v)
            hi_k, hi_v = plsc.sort_key_val(nk, nv, descending=True)
            sk, sv = plsc.sort_key_val(sk, sv, descending=True)
            rk2 = lax.rev(sk, (0,))
            rv2 = lax.rev(sv, (0,))
            m2 = lo_k >= rk2
            nk2 = jnp.where(m2, lo_k, rk2)
            nv2 = jnp.where(m2, lo_v, rv2)
            lo_k, lo_v = plsc.sort_key_val(nk2, nv2, descending=True)
        def unmap(kv):
            bd = ((kv - 2.0) * 0.5).astype(jnp.int32)
            return jnp.where(kv >= 2.0,
                             plsc.bitcast(_ONE_LO + bd, jnp.float32), kv)

        ovals_v[r, pl.ds(0, GW)] = unmap(hi_k)
        ovals_v[r, pl.ds(GW, GW)] = unmap(lo_k)
        oidx_v[r, pl.ds(0, GW)] = hi_v
        oidx_v[r, pl.ds(GW, GW)] = lo_v

    pltpu.sync_copy(ovals_v, vals_hbm.at[pl.ds(rb, rows_per_w)])
    pltpu.sync_copy(oidx_v, idx_hbm.at[pl.ds(rb, rows_per_w)])


def kernel(x, W, b):
    n, d_in = x.shape
    d_attr = W.shape[1]
    br = BR if n % BR == 0 else n
    nblk = n // br
    ng = n // GW

    w_pad = jnp.zeros((d_in, DP), jnp.float32).at[:, :d_attr].set(W)
    b_pad = jnp.full((1, DP), -1e30, jnp.float32).at[0, :d_attr].set(b)

    attr = pl.pallas_call(
        _attr_kernel,
        grid=(nblk,),
        in_specs=[
            pl.BlockSpec((br, d_in), lambda i: (i, 0)),
            pl.BlockSpec((d_in, DP), lambda i: (0, 0)),
            pl.BlockSpec((1, DP), lambda i: (0, 0)),
        ],
        out_specs=pl.BlockSpec((br, DP), lambda i: (i, 0)),
        out_shape=jax.ShapeDtypeStruct((n, DP), jnp.float32),
    )(x, w_pad, b_pad)

    maxval = pl.pallas_call(
        _maxval_kernel,
        grid=(nblk,),
        in_specs=[
            pl.BlockSpec((br, DP), lambda i: (i, 0)),
            pl.BlockSpec((n, DP), lambda i: (0, 0)),
        ],
        out_specs=pl.BlockSpec((1, n), lambda i: (0, 0)),
        out_shape=jax.ShapeDtypeStruct((1, n), jnp.float32),
    )(attr, attr)

    brg = BRG if n % BRG == 0 else n
    nblkg = n // brg
    table, gidx = pl.pallas_call(
        functools.partial(_groups_kernel, n=n, br=brg),
        grid=(nblkg,),
        in_specs=[
            pl.BlockSpec((brg, DP), lambda i: (i, 0)),
            pl.BlockSpec((n, DP), lambda i: (0, 0)),
            pl.BlockSpec((1, n), lambda i: (0, 0)),
        ],
        out_specs=[
            pl.BlockSpec((brg, n), lambda i: (i, 0)),
            pl.BlockSpec((brg, KPAD), lambda i: (i, 0)),
        ],
        out_shape=[
            jax.ShapeDtypeStruct((n, n), jnp.float32),
            jax.ShapeDtypeStruct((n, KPAD), jnp.int32),
        ],
    )(attr, attr, maxval)

    info = plsc.get_sparse_core_info()
    nc, ns = info.num_cores, info.num_subcores
    nw = nc * ns
    rows_per_w = n // nw

    sc_fn = pl.kernel(
        functools.partial(_sc_select_kernel, rows_per_w=rows_per_w, nc=nc,
                          ng=ng),
        out_type=[
            jax.ShapeDtypeStruct((n, 2 * GW), jnp.float32),
            jax.ShapeDtypeStruct((n, 2 * GW), jnp.int32),
        ],
        mesh=plsc.VectorSubcoreMesh(core_axis_name="c", subcore_axis_name="s"),
        compiler_params=pltpu.CompilerParams(needs_layout_passes=False),
        scratch_types=[
            pltpu.VMEM((rows_per_w, KPAD), jnp.int32),
            pltpu.VMEM((n,), jnp.float32),
            pltpu.VMEM((rows_per_w, 2 * GW), jnp.float32),
            pltpu.VMEM((rows_per_w, 2 * GW), jnp.int32),
            pltpu.SemaphoreType.DMA,
        ],
    )
    valsp, idxp = sc_fn(table, gidx)

    values = valsp[:, :K].reshape(-1)
    rows = jnp.repeat(jnp.arange(n, dtype=jnp.int32), K)
    edge_index = jnp.stack([rows, idxp[:, :K].reshape(-1)], axis=0)
    return values, edge_index


# speedup vs baseline: 15.2247x; 15.2247x over previous
import functools

import jax
import jax.numpy as jnp
from jax.experimental import pallas as pl
from jax.experimental.pallas import tpu as pltpu

K = 20
DP = 128
BR = 256
NEG = -3.0e38


def _attr_kernel(x_ref, w_ref, b_ref, out_ref):
    z = jnp.dot(x_ref[...], w_ref[...], preferred_element_type=jnp.float32)
    out_ref[...] = jax.nn.sigmoid(z + b_ref[...])


def _maxval_kernel(ab_ref, aall_ref, out_ref):
    a = jax.lax.dot_general(
        ab_ref[...], aall_ref[...],
        (((1,), (1,)), ((), ())),
        preferred_element_type=jnp.float32,
    )
    pmax = jnp.max(a, axis=0, keepdims=True)

    @pl.when(pl.program_id(0) == 0)
    def _():
        out_ref[...] = pmax

    @pl.when(pl.program_id(0) != 0)
    def _():
        out_ref[...] = jnp.maximum(out_ref[...], pmax)


def _topk_kernel(aall_ref, mv_ref, vals_ref, idx_ref, slab_ref, *, n, br, nblk):
    i = pl.program_id(0)

    @pl.when(i < nblk)
    def _():
        ab = aall_ref[pl.ds(i * br, br), :]
        a = jax.lax.dot_general(
            ab, aall_ref[...],
            (((1,), (1,)), ((), ())),
            preferred_element_type=jnp.float32,
        )
        off = (i % 2) * br
        slab_ref[pl.ds(off, br), :] = a / mv_ref[...]

    @pl.when(i > 0)
    def _():
        off = ((i - 1) % 2) * br
        iota = jax.lax.broadcasted_iota(jnp.int32, (br, n), 1)
        kiota = jax.lax.broadcasted_iota(jnp.int32, (br, K), 1)

        def body(k, acc):
            vacc, iacc = acc
            slab = slab_ref[pl.ds(off, br), :]
            m = jnp.max(slab, axis=1, keepdims=True)
            cand = jnp.where(slab == m, iota, n)
            ix = jnp.min(cand, axis=1, keepdims=True)
            slab_ref[pl.ds(off, br), :] = jnp.where(cand == ix, NEG, slab)
            vacc = jnp.where(kiota == k, m, vacc)
            iacc = jnp.where(kiota == k, ix, iacc)
            return vacc, iacc

        vacc, iacc = jax.lax.fori_loop(
            0, K, body,
            (jnp.zeros((br, K), jnp.float32), jnp.zeros((br, K), jnp.int32)))
        vals_ref[...] = vacc
        idx_ref[...] = iacc


def kernel(x, W, b):
    n, d_in = x.shape
    d_attr = W.shape[1]
    br = BR if n % BR == 0 else n
    nblk = n // br

    w_pad = jnp.zeros((d_in, DP), jnp.float32).at[:, :d_attr].set(W)
    b_pad = jnp.full((1, DP), -1e30, jnp.float32).at[0, :d_attr].set(b)

    attr = pl.pallas_call(
        _attr_kernel,
        grid=(nblk,),
        in_specs=[
            pl.BlockSpec((br, d_in), lambda i: (i, 0)),
            pl.BlockSpec((d_in, DP), lambda i: (0, 0)),
            pl.BlockSpec((1, DP), lambda i: (0, 0)),
        ],
        out_specs=pl.BlockSpec((br, DP), lambda i: (i, 0)),
        out_shape=jax.ShapeDtypeStruct((n, DP), jnp.float32),
    )(x, w_pad, b_pad)

    maxval = pl.pallas_call(
        _maxval_kernel,
        grid=(nblk,),
        in_specs=[
            pl.BlockSpec((br, DP), lambda i: (i, 0)),
            pl.BlockSpec((n, DP), lambda i: (0, 0)),
        ],
        out_specs=pl.BlockSpec((1, n), lambda i: (0, 0)),
        out_shape=jax.ShapeDtypeStruct((1, n), jnp.float32),
    )(attr, attr)

    vals, idxs = pl.pallas_call(
        functools.partial(_topk_kernel, n=n, br=br, nblk=nblk),
        grid=(nblk + 1,),
        in_specs=[
            pl.BlockSpec((n, DP), lambda i: (0, 0)),
            pl.BlockSpec((1, n), lambda i: (0, 0)),
        ],
        out_specs=[
            pl.BlockSpec((br, K), lambda i: (jnp.maximum(i - 1, 0), 0)),
            pl.BlockSpec((br, K), lambda i: (jnp.maximum(i - 1, 0), 0)),
        ],
        out_shape=[
            jax.ShapeDtypeStruct((n, K), jnp.float32),
            jax.ShapeDtypeStruct((n, K), jnp.int32),
        ],
        scratch_shapes=[pltpu.VMEM((2 * br, n), jnp.float32)],
    )(attr, maxval)

    values = vals.reshape(-1)
    rows = jnp.repeat(jnp.arange(n, dtype=jnp.int32), K)
    edge_index = jnp.stack([rows, idxs.reshape(-1)], axis=0)
    return values, edge_index
